# Initial kernel scaffold; baseline (speedup 1.0000x reference)
import functools

import jax
import jax.numpy as jnp
from jax import lax
from jax.experimental import pallas as pl
from jax.experimental.pallas import tpu as pltpu
from jax.experimental.pallas import tpu_sc as plsc

N_CHEM = 10000
N_GENE = 10000
E = 320000
D = 128
CHUNK = 128
NCHUNKS = E // CHUNK
NSUB = 16
ROWS_PER_TILE = N_GENE // NSUB
GATE_ROWS = 64
NBLK = 10
BLK = N_CHEM // NBLK


def _sigmoid(x):
    return 1.0 / (1.0 + jnp.exp(-x))


def _prep_body(xc, xg, wscg, bscg, wdcg, bdcg, wsgc, bsgc, wdgc, bdgc,
               cat, wccg, bccg, wcgc, bcgc,
               xs_cg, xd_cg, xs_gc, xd_gc, g_cg, g_gc):
    c = xc[...]
    g = xg[...]
    xs_cg[...] = jnp.dot(c, wscg[...], preferred_element_type=jnp.float32) + bscg[...]
    xd_cg[...] = jnp.dot(g, wdcg[...], preferred_element_type=jnp.float32) + bdcg[...]
    xs_gc[...] = jnp.dot(g, wsgc[...], preferred_element_type=jnp.float32) + bsgc[...]
    xd_gc[...] = jnp.dot(c, wdgc[...], preferred_element_type=jnp.float32) + bdgc[...]

    @pl.when(pl.program_id(0) == 0)
    def _():
        t = cat[...]
        g_cg[...] = _sigmoid(
            jnp.dot(t, wccg[...], preferred_element_type=jnp.float32) + bccg[...]) * 0.25
        g_gc[...] = _sigmoid(
            jnp.dot(t, wcgc[...], preferred_element_type=jnp.float32) + bcgc[...]) * 0.25


def _out_body(ac, woc, boc, ag, wog, bog, oc, og):
    oc[...] = jnp.dot(ac[...], woc[...], preferred_element_type=jnp.float32) + boc[...]
    og[...] = jnp.dot(ag[...], wog[...], preferred_element_type=jnp.float32) + bog[...]


def _sc_body(xs0, xd0, g0, si0, di0, code0,
             xs1, xd1, g1, si1, di1, code1,
             out_g, out_c,
             si_v, di_v, code_v, src_v, dst_v, gate_v, acc, sem):
    c = lax.axis_index("c")
    s = lax.axis_index("s")

    def _z(e, carry):
        for t in range(D // 16):
            src_v[e, pl.ds(t * 16, 16)] = jnp.zeros((16,), jnp.float32)
        return carry
    lax.fori_loop(0, CHUNK, _z, 0)
    for k in range(5):
        pltpu.sync_copy(src_v.at[pl.ds(0, 125)],
                        acc.at[pl.ds(s * ROWS_PER_TILE + k * 125, 125)])
    plsc.subcore_barrier()

    def _process(xs, xd, gt, si, di, code):
        n = jnp.where(s < NCHUNKS % NSUB, NCHUNKS // NSUB + 1, NCHUNKS // NSUB)

        def chunk_body(j, carry):
            base = (s + NSUB * j) * CHUNK
            cp0 = pltpu.async_copy(si.at[pl.ds(base, CHUNK)], si_v, sem)
            cp1 = pltpu.async_copy(di.at[pl.ds(base, CHUNK)], di_v, sem)
            cp2 = pltpu.async_copy(code.at[pl.ds(base, CHUNK)], code_v, sem)
            cp0.wait()
            cp1.wait()
            cp2.wait()
            gs = pltpu.async_copy(xs.at[si_v], src_v, sem)
            gd = pltpu.async_copy(xd.at[di_v], dst_v, sem)
            gg = pltpu.async_copy(gt.at[code_v], gate_v, sem)
            gs.wait()
            gd.wait()
            gg.wait()

            def e_body(e, carry2):
                for t in range(D // 16):
                    sl = pl.ds(t * 16, 16)
                    src_v[e, sl] = (src_v[e, sl] + dst_v[e, sl]) * gate_v[e, sl]
                return carry2
            lax.fori_loop(0, CHUNK, e_body, 0)
            pltpu.sync_copy(src_v, acc.at[di_v], add=True)
            return carry
        lax.fori_loop(0, n, chunk_body, 0)

    @pl.when(c == 0)
    def _():
        _process(xs0, xd0, g0, si0, di0, code0)

    @pl.when(c == 1)
    def _():
        _process(xs1, xd1, g1, si1, di1, code1)

    plsc.subcore_barrier()

    @pl.when(c == 0)
    def _():
        pltpu.sync_copy(acc.at[pl.ds(s * ROWS_PER_TILE, ROWS_PER_TILE)],
                        out_g.at[pl.ds(s * ROWS_PER_TILE, ROWS_PER_TILE)])

    @pl.when(c == 1)
    def _():
        pltpu.sync_copy(acc.at[pl.ds(s * ROWS_PER_TILE, ROWS_PER_TILE)],
                        out_c.at[pl.ds(s * ROWS_PER_TILE, ROWS_PER_TILE)])


_full128 = pl.BlockSpec((D, D), lambda i: (0, 0))
_full1x = pl.BlockSpec((1, D), lambda i: (0, 0))
_blk = pl.BlockSpec((BLK, D), lambda i: (i, 0))

_prep_call = pl.pallas_call(
    _prep_body,
    grid=(NBLK,),
    in_specs=[
        _blk, _blk,
        _full128, _full1x, _full128, _full1x,
        _full128, _full1x, _full128, _full1x,
        pl.BlockSpec((GATE_ROWS, GATE_ROWS), lambda i: (0, 0)),
        pl.BlockSpec((GATE_ROWS, D), lambda i: (0, 0)), _full1x,
        pl.BlockSpec((GATE_ROWS, D), lambda i: (0, 0)), _full1x,
    ],
    out_specs=[
        _blk, _blk, _blk, _blk,
        pl.BlockSpec((GATE_ROWS, D), lambda i: (0, 0)),
        pl.BlockSpec((GATE_ROWS, D), lambda i: (0, 0)),
    ],
    out_shape=[
        jax.ShapeDtypeStruct((N_CHEM, D), jnp.float32),
        jax.ShapeDtypeStruct((N_GENE, D), jnp.float32),
        jax.ShapeDtypeStruct((N_GENE, D), jnp.float32),
        jax.ShapeDtypeStruct((N_CHEM, D), jnp.float32),
        jax.ShapeDtypeStruct((GATE_ROWS, D), jnp.float32),
        jax.ShapeDtypeStruct((GATE_ROWS, D), jnp.float32),
    ],
)

_out_call = pl.pallas_call(
    _out_body,
    grid=(NBLK,),
    in_specs=[_blk, _full128, _full1x, _blk, _full128, _full1x],
    out_specs=[_blk, _blk],
    out_shape=[
        jax.ShapeDtypeStruct((N_CHEM, D), jnp.float32),
        jax.ShapeDtypeStruct((N_GENE, D), jnp.float32),
    ],
)

_sc_call = pl.kernel(
    _sc_body,
    out_type=[
        jax.ShapeDtypeStruct((N_GENE, D), jnp.float32),
        jax.ShapeDtypeStruct((N_CHEM, D), jnp.float32),
    ],
    mesh=plsc.VectorSubcoreMesh(core_axis_name="c", subcore_axis_name="s"),
    scratch_types=[
        pltpu.VMEM((CHUNK,), jnp.int32),
        pltpu.VMEM((CHUNK,), jnp.int32),
        pltpu.VMEM((CHUNK,), jnp.int32),
        pltpu.VMEM((CHUNK, D), jnp.float32),
        pltpu.VMEM((CHUNK, D), jnp.float32),
        pltpu.VMEM((CHUNK, D), jnp.float32),
        pltpu.VMEM_SHARED((N_GENE, D), jnp.float32),
        pltpu.SemaphoreType.DMA,
    ],
)


def kernel(x_chemical, x_gene, edge_index_cg, edge_index_gc, edge_attr_cg,
           edge_attr_gc, W_src_cg, b_src_cg, W_dst_cg, b_dst_cg, W_cat_cg,
           b_cat_cg, attn_cg, W_src_gc, b_src_gc, W_dst_gc, b_dst_gc,
           W_cat_gc, b_cat_gc, attn_gc, emb_action_type, emb_action_subject,
           W_out_chemical, b_out_chemical, W_out_gene, b_out_gene):
    num_as = emb_action_subject.shape[0]
    num_at = emb_action_type.shape[0]

    si_cg = edge_index_cg[0].astype(jnp.int32)
    di_cg = edge_index_cg[1].astype(jnp.int32)
    si_gc = edge_index_gc[0].astype(jnp.int32)
    di_gc = edge_index_gc[1].astype(jnp.int32)
    code_cg = (edge_attr_cg[:, 0] * num_as + edge_attr_cg[:, 1]).astype(jnp.int32)
    code_gc = (edge_attr_gc[:, 0] * num_as + edge_attr_gc[:, 1]).astype(jnp.int32)

    cat = jnp.concatenate(
        [jnp.repeat(emb_action_type, num_as, axis=0),
         jnp.tile(emb_action_subject, (num_at, 1))], axis=1)
    cat = jnp.pad(cat, ((0, GATE_ROWS - num_at * num_as), (0, 0)))

    xs_cg, xd_cg, xs_gc, xd_gc, g_cg, g_gc = _prep_call(
        x_chemical, x_gene,
        W_src_cg, b_src_cg.reshape(1, D), W_dst_cg, b_dst_cg.reshape(1, D),
        W_src_gc, b_src_gc.reshape(1, D), W_dst_gc, b_dst_gc.reshape(1, D),
        cat, W_cat_cg, b_cat_cg.reshape(1, D), W_cat_gc, b_cat_gc.reshape(1, D))

    aggr_gene, aggr_chem = _sc_call(
        xs_cg, xd_cg, g_cg, si_cg, di_cg, code_cg,
        xs_gc, xd_gc, g_gc, si_gc, di_gc, code_gc)

    out_chem, out_gene = _out_call(
        aggr_chem, W_out_chemical, b_out_chemical.reshape(1, D),
        aggr_gene, W_out_gene, b_out_gene.reshape(1, D))
    return (out_chem, out_gene)


# speedup vs baseline: 6.4287x; 6.4287x over previous
import functools

import jax
import jax.numpy as jnp
from jax import lax
from jax.experimental import pallas as pl
from jax.experimental.pallas import tpu as pltpu
from jax.experimental.pallas import tpu_sc as plsc

N_CHEM = 10000
N_GENE = 10000
E = 320000
D = 128
CHUNK = 80
NCHUNKS = E // CHUNK
NSUB = 16
NPAD = 10240
ZONE = NPAD // NSUB
GATE_ROWS = 64
NBLK = 10
BLK = N_CHEM // NBLK


def _sigmoid(x):
    return 1.0 / (1.0 + jnp.exp(-x))


def _prep_body(xc, xg, wscg, bscg, wdcg, bdcg, wsgc, bsgc, wdgc, bdgc,
               cat, wccg, bccg, wcgc, bcgc,
               xs_cg, xd_cg, xs_gc, xd_gc, g_cg, g_gc):
    c = xc[...]
    g = xg[...]
    xs_cg[...] = jnp.dot(c, wscg[...], preferred_element_type=jnp.float32) + bscg[...]
    xd_cg[...] = jnp.dot(g, wdcg[...], preferred_element_type=jnp.float32) + bdcg[...]
    xs_gc[...] = jnp.dot(g, wsgc[...], preferred_element_type=jnp.float32) + bsgc[...]
    xd_gc[...] = jnp.dot(c, wdgc[...], preferred_element_type=jnp.float32) + bdgc[...]

    @pl.when(pl.program_id(0) == 0)
    def _():
        t = cat[...]
        g_cg[...] = _sigmoid(
            jnp.dot(t, wccg[...], preferred_element_type=jnp.float32) + bccg[...]) * 0.25
        g_gc[...] = _sigmoid(
            jnp.dot(t, wcgc[...], preferred_element_type=jnp.float32) + bcgc[...]) * 0.25


def _out_body(ac, woc, boc, ag, wog, bog, oc, og):
    oc[...] = jnp.dot(ac[...], woc[...], preferred_element_type=jnp.float32) + boc[...]
    og[...] = jnp.dot(ag[...], wog[...], preferred_element_type=jnp.float32) + bog[...]


def _sc_body(xs0, xd0, g0, si0, di0, code0,
             xs1, xd1, g1, si1, di1, code1,
             out_g, out_c,
             si_v, di_v, code_v, src_v, dst_v, gate_v, acc, sem):
    c = lax.axis_index("c")
    s = lax.axis_index("s")

    def _z(e, carry):
        for t in range(D // 16):
            src_v[e, pl.ds(t * 16, 16)] = jnp.zeros((16,), jnp.float32)
        return carry
    lax.fori_loop(0, CHUNK, _z, 0)
    for k in range(ZONE // CHUNK):
        pltpu.sync_copy(src_v, acc.at[pl.ds(s * ZONE + k * CHUNK, CHUNK)])
    plsc.subcore_barrier()

    def _process(xs, xd, gt, si, di, code):
        n = jnp.where(s < NCHUNKS % NSUB, NCHUNKS // NSUB + 1, NCHUNKS // NSUB)

        def chunk_body(j, carry):
            base = (s + NSUB * j) * CHUNK
            cp0 = pltpu.async_copy(si.at[pl.ds(base, CHUNK)], si_v, sem)
            cp1 = pltpu.async_copy(di.at[pl.ds(base, CHUNK)], di_v, sem)
            cp2 = pltpu.async_copy(code.at[pl.ds(base, CHUNK)], code_v, sem)
            cp0.wait()
            cp1.wait()
            cp2.wait()
            gs = pltpu.async_copy(xs.at[si_v], src_v, sem)
            gd = pltpu.async_copy(xd.at[di_v], dst_v, sem)
            gg = pltpu.async_copy(gt.at[code_v], gate_v, sem)
            gs.wait()
            gd.wait()
            gg.wait()

            def e_body(e, carry2):
                for t in range(D // 16):
                    sl = pl.ds(t * 16, 16)
                    src_v[e, sl] = (src_v[e, sl] + dst_v[e, sl]) * gate_v[e, sl]
                return carry2
            lax.fori_loop(0, CHUNK, e_body, 0)
            pltpu.sync_copy(src_v, acc.at[di_v], add=True)
            return carry
        lax.fori_loop(0, n, chunk_body, 0)

    @pl.when(c == 0)
    def _():
        _process(xs0, xd0, g0, si0, di0, code0)

    @pl.when(c == 1)
    def _():
        _process(xs1, xd1, g1, si1, di1, code1)

    plsc.subcore_barrier()

    @pl.when(c == 0)
    def _():
        pltpu.sync_copy(acc.at[pl.ds(s * ZONE, ZONE)],
                        out_g.at[pl.ds(s * ZONE, ZONE)])

    @pl.when(c == 1)
    def _():
        pltpu.sync_copy(acc.at[pl.ds(s * ZONE, ZONE)],
                        out_c.at[pl.ds(s * ZONE, ZONE)])


_full128 = pl.BlockSpec((D, D), lambda i: (0, 0))
_full1x = pl.BlockSpec((1, D), lambda i: (0, 0))
_blk = pl.BlockSpec((BLK, D), lambda i: (i, 0))

_prep_call = pl.pallas_call(
    _prep_body,
    grid=(NBLK,),
    in_specs=[
        _blk, _blk,
        _full128, _full1x, _full128, _full1x,
        _full128, _full1x, _full128, _full1x,
        pl.BlockSpec((GATE_ROWS, GATE_ROWS), lambda i: (0, 0)),
        pl.BlockSpec((GATE_ROWS, D), lambda i: (0, 0)), _full1x,
        pl.BlockSpec((GATE_ROWS, D), lambda i: (0, 0)), _full1x,
    ],
    out_specs=[
        _blk, _blk, _blk, _blk,
        pl.BlockSpec((GATE_ROWS, D), lambda i: (0, 0)),
        pl.BlockSpec((GATE_ROWS, D), lambda i: (0, 0)),
    ],
    out_shape=[
        jax.ShapeDtypeStruct((N_CHEM, D), jnp.float32),
        jax.ShapeDtypeStruct((N_GENE, D), jnp.float32),
        jax.ShapeDtypeStruct((N_GENE, D), jnp.float32),
        jax.ShapeDtypeStruct((N_CHEM, D), jnp.float32),
        jax.ShapeDtypeStruct((GATE_ROWS, D), jnp.float32),
        jax.ShapeDtypeStruct((GATE_ROWS, D), jnp.float32),
    ],
)

_out_call = pl.pallas_call(
    _out_body,
    grid=(NBLK,),
    in_specs=[_blk, _full128, _full1x, _blk, _full128, _full1x],
    out_specs=[_blk, _blk],
    out_shape=[
        jax.ShapeDtypeStruct((N_CHEM, D), jnp.float32),
        jax.ShapeDtypeStruct((N_GENE, D), jnp.float32),
    ],
)

_sc_call = pl.kernel(
    _sc_body,
    out_type=[
        jax.ShapeDtypeStruct((NPAD, D), jnp.float32),
        jax.ShapeDtypeStruct((NPAD, D), jnp.float32),
    ],
    mesh=plsc.VectorSubcoreMesh(core_axis_name="c", subcore_axis_name="s"),
    scratch_types=[
        pltpu.VMEM((CHUNK,), jnp.int32),
        pltpu.VMEM((CHUNK,), jnp.int32),
        pltpu.VMEM((CHUNK,), jnp.int32),
        pltpu.VMEM((CHUNK, D), jnp.float32),
        pltpu.VMEM((CHUNK, D), jnp.float32),
        pltpu.VMEM((CHUNK, D), jnp.float32),
        pltpu.VMEM_SHARED((NPAD, D), jnp.float32),
        pltpu.SemaphoreType.DMA,
    ],
)


def kernel(x_chemical, x_gene, edge_index_cg, edge_index_gc, edge_attr_cg,
           edge_attr_gc, W_src_cg, b_src_cg, W_dst_cg, b_dst_cg, W_cat_cg,
           b_cat_cg, attn_cg, W_src_gc, b_src_gc, W_dst_gc, b_dst_gc,
           W_cat_gc, b_cat_gc, attn_gc, emb_action_type, emb_action_subject,
           W_out_chemical, b_out_chemical, W_out_gene, b_out_gene):
    num_as = emb_action_subject.shape[0]
    num_at = emb_action_type.shape[0]

    si_cg = edge_index_cg[0].astype(jnp.int32)
    di_cg = edge_index_cg[1].astype(jnp.int32)
    si_gc = edge_index_gc[0].astype(jnp.int32)
    di_gc = edge_index_gc[1].astype(jnp.int32)
    code_cg = (edge_attr_cg[:, 0] * num_as + edge_attr_cg[:, 1]).astype(jnp.int32)
    code_gc = (edge_attr_gc[:, 0] * num_as + edge_attr_gc[:, 1]).astype(jnp.int32)

    cat = jnp.concatenate(
        [jnp.repeat(emb_action_type, num_as, axis=0),
         jnp.tile(emb_action_subject, (num_at, 1))], axis=1)
    cat = jnp.pad(cat, ((0, GATE_ROWS - num_at * num_as), (0, 0)))

    xs_cg, xd_cg, xs_gc, xd_gc, g_cg, g_gc = _prep_call(
        x_chemical, x_gene,
        W_src_cg, b_src_cg.reshape(1, D), W_dst_cg, b_dst_cg.reshape(1, D),
        W_src_gc, b_src_gc.reshape(1, D), W_dst_gc, b_dst_gc.reshape(1, D),
        cat, W_cat_cg, b_cat_cg.reshape(1, D), W_cat_gc, b_cat_gc.reshape(1, D))

    aggr_gene_p, aggr_chem_p = _sc_call(
        xs_cg, xd_cg, g_cg, si_cg, di_cg, code_cg,
        xs_gc, xd_gc, g_gc, si_gc, di_gc, code_gc)
    aggr_gene = aggr_gene_p[:N_GENE]
    aggr_chem = aggr_chem_p[:N_CHEM]

    out_chem, out_gene = _out_call(
        aggr_chem, W_out_chemical, b_out_chemical.reshape(1, D),
        aggr_gene, W_out_gene, b_out_gene.reshape(1, D))
    return (out_chem, out_gene)
